# Initial kernel scaffold; baseline (speedup 1.0000x reference)
import jax
import jax.numpy as jnp
from jax.experimental import pallas as pl


def kernel(edge_features, W0, W1, W2):
    raise NotImplementedError("write your pallas kernel here")


# speedup vs baseline: 1.4717x; 1.4717x over previous
import functools

import jax
import jax.numpy as jnp
from jax import lax
from jax.experimental import pallas as pl
from jax.experimental.pallas import tpu as pltpu
from jax.experimental.pallas import tpu_sc as plsc

V0, V1, V2 = 12, 15, 7
D = 32
NROWS = V0 * V1 * V2
NPAD = 1280
NC, NS = 2, 16
NW = NC * NS


def _fuse_body(w0_ref, w1_ref, w2_ref, out_ref):
    r0 = lax.broadcasted_iota(jnp.int32, (NPAD, V0), 0)
    k0 = lax.broadcasted_iota(jnp.int32, (NPAD, V0), 1)
    oh0 = (r0 % V0 == k0).astype(jnp.float32)
    r1 = lax.broadcasted_iota(jnp.int32, (NPAD, V1), 0)
    k1 = lax.broadcasted_iota(jnp.int32, (NPAD, V1), 1)
    oh1 = ((r1 // V0) % V1 == k1).astype(jnp.float32)
    r2 = lax.broadcasted_iota(jnp.int32, (NPAD, V2), 0)
    k2 = lax.broadcasted_iota(jnp.int32, (NPAD, V2), 1)
    oh2 = (r2 // (V0 * V1) == k2).astype(jnp.float32)
    out_ref[...] = (
        jnp.dot(oh0, w0_ref[...], preferred_element_type=jnp.float32,
                  precision=lax.Precision.HIGHEST)
        + jnp.dot(oh1, w1_ref[...], preferred_element_type=jnp.float32,
                  precision=lax.Precision.HIGHEST)
        + jnp.dot(oh2, w2_ref[...], preferred_element_type=jnp.float32,
                  precision=lax.Precision.HIGHEST)
    )


def _build_fused(W0, W1, W2):
    return pl.pallas_call(
        _fuse_body,
        out_shape=jax.ShapeDtypeStruct((NPAD, D), jnp.float32),
    )(W0, W1, W2)


@functools.partial(jax.jit, static_argnames=("E", "B"))
def _sc_gather(wf_flat, ef_flat, E, B):
    EW = E // NW
    NBLK = EW // B
    G = B // 16

    mesh = plsc.VectorSubcoreMesh(
        core_axis_name="c", subcore_axis_name="s",
        num_cores=NC, num_subcores=NS)

    def body(wf_hbm, ef_hbm, out_hbm, wf_v, ef_v, out_v):
        wid = lax.axis_index("s") * NC + lax.axis_index("c")
        pltpu.sync_copy(wf_hbm, wf_v)
        lane = lax.iota(jnp.int32, 16)
        lane3 = lane * 3
        lane32 = lane * D
        e_base = wid * EW

        def block(b, _):
            e0 = e_base + b * B
            pltpu.sync_copy(ef_hbm.at[pl.ds(e0 * 3, B * 3)], ef_v)

            def grp(j, _):
                i3 = j * 48 + lane3
                g0 = plsc.load_gather(ef_v, [i3])
                g1 = plsc.load_gather(ef_v, [i3 + 1])
                g2 = plsc.load_gather(ef_v, [i3 + 2])
                bidx = (g0 + g1 * V0 + g2 * (V0 * V1)) * D
                sidx = j * (16 * D) + lane32
                for d in range(D):
                    v = plsc.load_gather(wf_v, [bidx + d])
                    plsc.store_scatter(out_v, [sidx + d], v)
                return 0

            lax.fori_loop(0, G, grp, 0, unroll=False)
            pltpu.sync_copy(out_v, out_hbm.at[pl.ds(e0 * D, B * D)])
            return 0

        lax.fori_loop(0, NBLK, block, 0, unroll=False)

    return pl.kernel(
        body,
        out_type=jax.ShapeDtypeStruct((E * D,), jnp.float32),
        mesh=mesh,
        compiler_params=pltpu.CompilerParams(needs_layout_passes=False),
        scratch_types=[
            pltpu.VMEM((NPAD * D,), jnp.float32),
            pltpu.VMEM((B * 3,), jnp.int32),
            pltpu.VMEM((B * D,), jnp.float32),
        ],
    )(wf_flat, ef_flat)


def _pick_chunk(EW):
    best = 16
    for cand in range(16, 1025, 16):
        if EW % cand == 0:
            best = cand
    return best


def kernel(edge_features, W0, W1, W2):
    E = edge_features.shape[0]
    ef = edge_features.astype(jnp.int32).reshape(-1)
    wf = _build_fused(W0, W1, W2).reshape(-1)
    out = _sc_gather(wf, ef, E, _pick_chunk(E // NW))
    return out.reshape(E, D)


# speedup vs baseline: 1.7853x; 1.2130x over previous
import functools

import jax
import jax.numpy as jnp
from jax import lax
from jax.experimental import pallas as pl
from jax.experimental.pallas import tpu as pltpu
from jax.experimental.pallas import tpu_sc as plsc

V0, V1, V2 = 12, 15, 7
D = 32
NROWS = V0 * V1 * V2
NPAD = 1280
NC, NS = 2, 16
NW = NC * NS


def _fuse_body(w0_ref, w1_ref, w2_ref, out_ref):
    r0 = lax.broadcasted_iota(jnp.int32, (NPAD, V0), 0)
    k0 = lax.broadcasted_iota(jnp.int32, (NPAD, V0), 1)
    oh0 = (r0 % V0 == k0).astype(jnp.float32)
    r1 = lax.broadcasted_iota(jnp.int32, (NPAD, V1), 0)
    k1 = lax.broadcasted_iota(jnp.int32, (NPAD, V1), 1)
    oh1 = ((r1 // V0) % V1 == k1).astype(jnp.float32)
    r2 = lax.broadcasted_iota(jnp.int32, (NPAD, V2), 0)
    k2 = lax.broadcasted_iota(jnp.int32, (NPAD, V2), 1)
    oh2 = (r2 // (V0 * V1) == k2).astype(jnp.float32)
    out_ref[...] = (
        jnp.dot(oh0, w0_ref[...], preferred_element_type=jnp.float32,
                precision=lax.Precision.HIGHEST)
        + jnp.dot(oh1, w1_ref[...], preferred_element_type=jnp.float32,
                  precision=lax.Precision.HIGHEST)
        + jnp.dot(oh2, w2_ref[...], preferred_element_type=jnp.float32,
                  precision=lax.Precision.HIGHEST)
    )


def _build_fused(W0, W1, W2):
    return pl.pallas_call(
        _fuse_body,
        out_shape=jax.ShapeDtypeStruct((NPAD, D), jnp.float32),
    )(W0, W1, W2)


@functools.partial(jax.jit, static_argnames=("E", "S"))
def _sc_gather(wf_flat, ef_t, E, S):
    EW = E // NW
    NCHUNK = EW // S
    G = S // 16

    mesh = plsc.VectorSubcoreMesh(
        core_axis_name="c", subcore_axis_name="s",
        num_cores=NC, num_subcores=NS)

    def body(wf_hbm, ef_hbm, out_hbm, wf_v, e0_v, e1_v, e2_v, c_v, out_v,
             sem_out):
        wid = lax.axis_index("s") * NC + lax.axis_index("c")
        pltpu.sync_copy(wf_hbm, wf_v)
        e_base = wid * EW

        def chunk(t, _):
            e0 = e_base + t * S
            pltpu.sync_copy(ef_hbm.at[pl.ds(e0, S)], e0_v)
            pltpu.sync_copy(ef_hbm.at[pl.ds(E + e0, S)], e1_v)
            pltpu.sync_copy(ef_hbm.at[pl.ds(2 * E + e0, S)], e2_v)

            @plsc.parallel_loop(0, G, unroll=5)
            def _(j):
                sl = pl.ds(j * 16, 16)
                c_v[sl] = (e0_v[sl] + e1_v[sl] * V0 + e2_v[sl] * (V0 * V1)) * D

            @plsc.parallel_loop(0, G, unroll=5)
            def _(j):
                base = c_v[pl.ds(j * 16, 16)]
                for d in range(D):
                    v = plsc.load_gather(wf_v, [base + d])
                    out_v[pl.ds(d * S + j * 16, 16)] = v

            copies = [
                pltpu.async_copy(
                    out_v.at[pl.ds(d * S, S)],
                    out_hbm.at[pl.ds(d * E + e0, S)],
                    sem_out)
                for d in range(D)
            ]
            for c in copies:
                c.wait()
            return 0

        lax.fori_loop(0, NCHUNK, chunk, 0, unroll=False)

    return pl.kernel(
        body,
        out_type=jax.ShapeDtypeStruct((D * E,), jnp.float32),
        mesh=mesh,
        compiler_params=pltpu.CompilerParams(needs_layout_passes=False),
        scratch_types=[
            pltpu.VMEM((NPAD * D,), jnp.float32),
            pltpu.VMEM((S,), jnp.int32),
            pltpu.VMEM((S,), jnp.int32),
            pltpu.VMEM((S,), jnp.int32),
            pltpu.VMEM((S,), jnp.int32),
            pltpu.VMEM((D * S,), jnp.float32),
            pltpu.SemaphoreType.DMA,
        ],
    )(wf_flat, ef_t)


def _pick_chunk(EW):
    best = 16
    for cand in range(16, 2049, 16):
        if EW % cand == 0:
            best = cand
    return best


def kernel(edge_features, W0, W1, W2):
    E = edge_features.shape[0]
    ef_t = edge_features.astype(jnp.int32).T.reshape(-1)
    wf = _build_fused(W0, W1, W2).reshape(-1)
    out = _sc_gather(wf, ef_t, E, _pick_chunk(E // NW))
    return out.reshape(D, E).T


# speedup vs baseline: 1.8770x; 1.0514x over previous
import functools

import jax
import jax.numpy as jnp
from jax import lax
from jax.experimental import pallas as pl
from jax.experimental.pallas import tpu as pltpu
from jax.experimental.pallas import tpu_sc as plsc

V0, V1, V2 = 12, 15, 7
D = 32
NROWS = V0 * V1 * V2
NPAD = 1280
NC, NS = 2, 16
NW = NC * NS


def _fuse_body(w0_ref, w1_ref, w2_ref, out_ref):
    r0 = lax.broadcasted_iota(jnp.int32, (NPAD, V0), 0)
    k0 = lax.broadcasted_iota(jnp.int32, (NPAD, V0), 1)
    oh0 = (r0 % V0 == k0).astype(jnp.float32)
    r1 = lax.broadcasted_iota(jnp.int32, (NPAD, V1), 0)
    k1 = lax.broadcasted_iota(jnp.int32, (NPAD, V1), 1)
    oh1 = ((r1 // V0) % V1 == k1).astype(jnp.float32)
    r2 = lax.broadcasted_iota(jnp.int32, (NPAD, V2), 0)
    k2 = lax.broadcasted_iota(jnp.int32, (NPAD, V2), 1)
    oh2 = (r2 // (V0 * V1) == k2).astype(jnp.float32)
    out_ref[...] = (
        jnp.dot(oh0, w0_ref[...], preferred_element_type=jnp.float32,
                precision=lax.Precision.HIGHEST)
        + jnp.dot(oh1, w1_ref[...], preferred_element_type=jnp.float32,
                  precision=lax.Precision.HIGHEST)
        + jnp.dot(oh2, w2_ref[...], preferred_element_type=jnp.float32,
                  precision=lax.Precision.HIGHEST)
    )


def _build_fused(W0, W1, W2):
    return pl.pallas_call(
        _fuse_body,
        out_shape=jax.ShapeDtypeStruct((NPAD, D), jnp.float32),
    )(W0, W1, W2)


@functools.partial(jax.jit, static_argnames=("E", "S"))
def _sc_gather(wf_flat, c32, E, S):
    EW = E // NW
    NCHUNK = EW // S
    G = S // 16

    mesh = plsc.VectorSubcoreMesh(
        core_axis_name="c", subcore_axis_name="s",
        num_cores=NC, num_subcores=NS)

    def body(wf_hbm, c_hbm, out_hbm, wf_v, c_v, out_v, sem_out):
        wid = lax.axis_index("s") * NC + lax.axis_index("c")
        pltpu.sync_copy(wf_hbm, wf_v)
        e_base = wid * EW

        def chunk(t, _):
            e0 = e_base + t * S
            pltpu.sync_copy(c_hbm.at[pl.ds(e0, S)], c_v)

            @plsc.parallel_loop(0, G, unroll=5)
            def _(j):
                base = c_v[pl.ds(j * 16, 16)]
                for d in range(D):
                    v = plsc.load_gather(wf_v, [base + d])
                    out_v[pl.ds(d * S + j * 16, 16)] = v

            copies = [
                pltpu.async_copy(
                    out_v.at[pl.ds(d * S, S)],
                    out_hbm.at[pl.ds(d * E + e0, S)],
                    sem_out)
                for d in range(D)
            ]
            for c in copies:
                c.wait()
            return 0

        lax.fori_loop(0, NCHUNK, chunk, 0, unroll=False)

    return pl.kernel(
        body,
        out_type=jax.ShapeDtypeStruct((D * E,), jnp.float32),
        mesh=mesh,
        compiler_params=pltpu.CompilerParams(needs_layout_passes=False),
        scratch_types=[
            pltpu.VMEM((NPAD * D,), jnp.float32),
            pltpu.VMEM((S,), jnp.int32),
            pltpu.VMEM((D * S,), jnp.float32),
            pltpu.SemaphoreType.DMA,
        ],
    )(wf_flat, c32)


def _pick_chunk(EW):
    best = 16
    for cand in range(16, 2049, 16):
        if EW % cand == 0:
            best = cand
    return best


def kernel(edge_features, W0, W1, W2):
    E = edge_features.shape[0]
    ef = edge_features.astype(jnp.int32)
    c32 = (ef[:, 0] + ef[:, 1] * V0 + ef[:, 2] * (V0 * V1)) * D
    wf = _build_fused(W0, W1, W2).reshape(-1)
    out = _sc_gather(wf, c32, E, _pick_chunk(E // NW))
    return out.reshape(D, E).T


# speedup vs baseline: 13.1179x; 6.9887x over previous
import functools

import jax
import jax.numpy as jnp
from jax import lax
from jax.experimental import pallas as pl
from jax.experimental.pallas import tpu as pltpu
from jax.experimental.pallas import tpu_sc as plsc

V0, V1, V2 = 12, 15, 7
D = 32
NROWS = V0 * V1 * V2
NPAD = 1280
NC, NS = 2, 16
NW = NC * NS


def _fuse_body(w0_ref, w1_ref, w2_ref, out_ref):
    r0 = lax.broadcasted_iota(jnp.int32, (NPAD, V0), 0)
    k0 = lax.broadcasted_iota(jnp.int32, (NPAD, V0), 1)
    oh0 = (r0 % V0 == k0).astype(jnp.float32)
    r1 = lax.broadcasted_iota(jnp.int32, (NPAD, V1), 0)
    k1 = lax.broadcasted_iota(jnp.int32, (NPAD, V1), 1)
    oh1 = ((r1 // V0) % V1 == k1).astype(jnp.float32)
    r2 = lax.broadcasted_iota(jnp.int32, (NPAD, V2), 0)
    k2 = lax.broadcasted_iota(jnp.int32, (NPAD, V2), 1)
    oh2 = (r2 // (V0 * V1) == k2).astype(jnp.float32)
    out_ref[...] = (
        jnp.dot(oh0, w0_ref[...], preferred_element_type=jnp.float32,
                precision=lax.Precision.HIGHEST)
        + jnp.dot(oh1, w1_ref[...], preferred_element_type=jnp.float32,
                  precision=lax.Precision.HIGHEST)
        + jnp.dot(oh2, w2_ref[...], preferred_element_type=jnp.float32,
                  precision=lax.Precision.HIGHEST)
    )


def _build_fused(W0, W1, W2):
    return pl.pallas_call(
        _fuse_body,
        out_shape=jax.ShapeDtypeStruct((NPAD, D), jnp.float32),
    )(W0, W1, W2)


@functools.partial(jax.jit, static_argnames=("E", "S"))
def _sc_gather(wf_flat, c32, E, S):
    TL = S // 128
    NCH = E // S

    mesh = plsc.VectorSubcoreMesh(
        core_axis_name="c", subcore_axis_name="s",
        num_cores=NC, num_subcores=NS)

    def body(wf_hbm, c_hbm, out_hbm, wf_v, c_v, out_v, sem_out):
        wid = lax.axis_index("s") * NC + lax.axis_index("c")
        pltpu.sync_copy(wf_hbm, wf_v)
        cnt = (NCH - wid + NW - 1) // NW

        def chunk(t, _):
            ch = wid + t * NW
            e0 = ch * S
            pltpu.sync_copy(c_hbm.at[pl.ds(e0, S)], c_v)

            @plsc.parallel_loop(0, TL, unroll=1)
            def _(tl):
                cvecs = [c_v[pl.ds(tl * 128 + q * 16, 16)] for q in range(8)]
                for d in range(D):
                    o = (d // 8) * (TL * 1024) + tl * 1024 + (d % 8) * 128
                    for q in range(8):
                        v = plsc.load_gather(wf_v, [cvecs[q] + d])
                        out_v[pl.ds(o + q * 16, 16)] = v

            copies = [
                pltpu.async_copy(
                    out_v.at[pl.ds(dg * (TL * 1024), TL * 1024)],
                    out_hbm.at[pl.ds(dg * (8 * E) + ch * (TL * 1024),
                                     TL * 1024)],
                    sem_out)
                for dg in range(4)
            ]
            for c in copies:
                c.wait()
            return 0

        lax.fori_loop(0, cnt, chunk, 0, unroll=False)

    return pl.kernel(
        body,
        out_type=jax.ShapeDtypeStruct((D * E,), jnp.float32),
        mesh=mesh,
        compiler_params=pltpu.CompilerParams(needs_layout_passes=False),
        scratch_types=[
            pltpu.VMEM((NPAD * D,), jnp.float32),
            pltpu.VMEM((S,), jnp.int32),
            pltpu.VMEM((4 * TL * 1024,), jnp.float32),
            pltpu.SemaphoreType.DMA,
        ],
    )(wf_flat, c32)


def kernel(edge_features, W0, W1, W2):
    E = edge_features.shape[0]
    S = 1280
    assert E % S == 0
    ef = edge_features.astype(jnp.int32)
    c32 = (ef[:, 0] + ef[:, 1] * V0 + ef[:, 2] * (V0 * V1)) * D
    wf = _build_fused(W0, W1, W2).reshape(-1)
    out = _sc_gather(wf, c32, E, S)
    return out.reshape(4, E // 128, 8, 128).transpose(1, 3, 0, 2).reshape(E, D)


# speedup vs baseline: 14.4452x; 1.1012x over previous
import functools

import jax
import jax.numpy as jnp
from jax import lax
from jax.experimental import pallas as pl
from jax.experimental.pallas import tpu as pltpu
from jax.experimental.pallas import tpu_sc as plsc

V0, V1, V2 = 12, 15, 7
D = 32
NROWS = V0 * V1 * V2
NPAD = 1280
NC, NS = 2, 16
NW = NC * NS


def _fuse_body(w0_ref, w1_ref, w2_ref, out_ref):
    r0 = lax.broadcasted_iota(jnp.int32, (NPAD, V0), 0)
    k0 = lax.broadcasted_iota(jnp.int32, (NPAD, V0), 1)
    oh0 = (r0 % V0 == k0).astype(jnp.float32)
    r1 = lax.broadcasted_iota(jnp.int32, (NPAD, V1), 0)
    k1 = lax.broadcasted_iota(jnp.int32, (NPAD, V1), 1)
    oh1 = ((r1 // V0) % V1 == k1).astype(jnp.float32)
    r2 = lax.broadcasted_iota(jnp.int32, (NPAD, V2), 0)
    k2 = lax.broadcasted_iota(jnp.int32, (NPAD, V2), 1)
    oh2 = (r2 // (V0 * V1) == k2).astype(jnp.float32)
    out_ref[...] = (
        jnp.dot(oh0, w0_ref[...], preferred_element_type=jnp.float32,
                precision=lax.Precision.HIGHEST)
        + jnp.dot(oh1, w1_ref[...], preferred_element_type=jnp.float32,
                  precision=lax.Precision.HIGHEST)
        + jnp.dot(oh2, w2_ref[...], preferred_element_type=jnp.float32,
                  precision=lax.Precision.HIGHEST)
    )


def _build_fused(W0, W1, W2):
    return pl.pallas_call(
        _fuse_body,
        out_shape=jax.ShapeDtypeStruct((NPAD, D), jnp.float32),
    )(W0, W1, W2)


@functools.partial(jax.jit, static_argnames=("E", "S"))
def _sc_gather(wf_flat, c32, E, S):
    TL = S // 128
    NCH = E // S

    mesh = plsc.VectorSubcoreMesh(
        core_axis_name="c", subcore_axis_name="s",
        num_cores=NC, num_subcores=NS)

    OSZ = 4 * TL * 1024

    def body(wf_hbm, c_hbm, out_hbm, wf_v, c_v, out_v, sem_c, sem_out):
        wid = lax.axis_index("s") * NC + lax.axis_index("c")
        pltpu.sync_copy(wf_hbm, wf_v)
        cnt = (NCH - wid + NW - 1) // NW

        def c_copy(t):
            tt = jnp.minimum(t, cnt - 1)
            ch = wid + tt * NW
            return pltpu.async_copy(
                c_hbm.at[pl.ds(ch * S, S)],
                c_v.at[pl.ds((t % 2) * S, S)],
                sem_c)

        c_copy(0).wait()

        def chunk(t, _):
            p = t % 2
            ch = wid + t * NW
            c_copy(t + 1)

            cbase = p * S
            obase = p * OSZ

            @plsc.parallel_loop(0, TL, unroll=1)
            def _(tl):
                cvecs = [c_v[pl.ds(cbase + tl * 128 + q * 16, 16)]
                         for q in range(8)]
                for d in range(D):
                    o = obase + (d // 8) * (TL * 1024) + tl * 1024 \
                        + (d % 8) * 128
                    for q in range(8):
                        v = plsc.load_gather(wf_v, [cvecs[q] + d])
                        out_v[pl.ds(o + q * 16, 16)] = v

            copies = [
                pltpu.async_copy(
                    out_v.at[pl.ds(obase + dg * (TL * 1024), TL * 1024)],
                    out_hbm.at[pl.ds(dg * (8 * E) + ch * (TL * 1024),
                                     TL * 1024)],
                    sem_out)
                for dg in range(4)
            ]

            @pl.when(t > 0)
            def _():
                for _ in range(4):
                    pltpu.make_async_copy(
                        out_v.at[pl.ds(0, TL * 1024)],
                        out_hbm.at[pl.ds(0, TL * 1024)],
                        sem_out).wait()

            pltpu.make_async_copy(
                c_hbm.at[pl.ds(0, S)], c_v.at[pl.ds(0, S)], sem_c).wait()
            return 0

        lax.fori_loop(0, cnt, chunk, 0, unroll=False)
        for _ in range(4):
            pltpu.make_async_copy(
                out_v.at[pl.ds(0, TL * 1024)],
                out_hbm.at[pl.ds(0, TL * 1024)],
                sem_out).wait()

    return pl.kernel(
        body,
        out_type=jax.ShapeDtypeStruct((D * E,), jnp.float32),
        mesh=mesh,
        compiler_params=pltpu.CompilerParams(needs_layout_passes=False),
        scratch_types=[
            pltpu.VMEM((NPAD * D,), jnp.float32),
            pltpu.VMEM((2 * S,), jnp.int32),
            pltpu.VMEM((2 * 4 * TL * 1024,), jnp.float32),
            pltpu.SemaphoreType.DMA,
            pltpu.SemaphoreType.DMA,
        ],
    )(wf_flat, c32)


def kernel(edge_features, W0, W1, W2):
    E = edge_features.shape[0]
    S = 1280
    assert E % S == 0
    ef = edge_features.astype(jnp.int32)
    c32 = (ef[:, 0] + ef[:, 1] * V0 + ef[:, 2] * (V0 * V1)) * D
    wf = _build_fused(W0, W1, W2).reshape(-1)
    out = _sc_gather(wf, c32, E, S)
    return out.reshape(4, E // 128, 8, 128).transpose(1, 3, 0, 2).reshape(E, D)


# speedup vs baseline: 17.8672x; 1.2369x over previous
import functools

import jax
import jax.numpy as jnp
from jax import lax
from jax.experimental import pallas as pl
from jax.experimental.pallas import tpu as pltpu
from jax.experimental.pallas import tpu_sc as plsc

V0, V1, V2 = 12, 15, 7
D = 32
NROWS = V0 * V1 * V2
NPAD = 1280
NC, NS = 2, 16
NW = NC * NS


def _fuse_body(w0_ref, w1_ref, w2_ref, out_ref):
    r0 = lax.broadcasted_iota(jnp.int32, (NPAD, V0), 0)
    k0 = lax.broadcasted_iota(jnp.int32, (NPAD, V0), 1)
    oh0 = (r0 % V0 == k0).astype(jnp.float32)
    r1 = lax.broadcasted_iota(jnp.int32, (NPAD, V1), 0)
    k1 = lax.broadcasted_iota(jnp.int32, (NPAD, V1), 1)
    oh1 = ((r1 // V0) % V1 == k1).astype(jnp.float32)
    r2 = lax.broadcasted_iota(jnp.int32, (NPAD, V2), 0)
    k2 = lax.broadcasted_iota(jnp.int32, (NPAD, V2), 1)
    oh2 = (r2 // (V0 * V1) == k2).astype(jnp.float32)
    out_ref[...] = (
        jnp.dot(oh0, w0_ref[...], preferred_element_type=jnp.float32,
                precision=lax.Precision.HIGHEST)
        + jnp.dot(oh1, w1_ref[...], preferred_element_type=jnp.float32,
                  precision=lax.Precision.HIGHEST)
        + jnp.dot(oh2, w2_ref[...], preferred_element_type=jnp.float32,
                  precision=lax.Precision.HIGHEST)
    )


def _build_fused(W0, W1, W2):
    return pl.pallas_call(
        _fuse_body,
        out_shape=jax.ShapeDtypeStruct((NPAD, D), jnp.float32),
    )(W0, W1, W2)


@functools.partial(jax.jit, static_argnames=("E", "S"))
def _sc_gather(wf_flat, c32, E, S):
    TL = S // 128
    NCH = E // S

    mesh = plsc.VectorSubcoreMesh(
        core_axis_name="c", subcore_axis_name="s",
        num_cores=NC, num_subcores=NS)

    OSZ = 4 * TL * 1024

    def body(wf_hbm, c_hbm, out_hbm, wf_v, c_v, out_v, sem_c, sem_out):
        wid = lax.axis_index("s") * NC + lax.axis_index("c")
        lane = lax.iota(jnp.int32, 16)
        pltpu.sync_copy(wf_hbm, out_v.at[pl.ds(0, NPAD * D)])

        @plsc.parallel_loop(0, NROWS, unroll=4)
        def _(c):
            for h in (0, 1):
                src = out_v[pl.ds(c * D + h * 16, 16)]
                dst = c * D + h * 16 + ((lane + c) & 15)
                plsc.store_scatter(wf_v, [dst], src)

        cnt = (NCH - wid + NW - 1) // NW

        def c_copy(t):
            tt = jnp.minimum(t, cnt - 1)
            ch = wid + tt * NW
            return pltpu.async_copy(
                c_hbm.at[pl.ds(ch * S, S)],
                c_v.at[pl.ds((t % 2) * S, S)],
                sem_c)

        c_copy(0).wait()

        def chunk(t, _):
            p = t % 2
            ch = wid + t * NW
            c_copy(t + 1)

            cbase = p * S
            obase = p * OSZ

            @plsc.parallel_loop(0, TL, unroll=1)
            def _(tl):
                cvecs = []
                for q in range(8):
                    c32 = c_v[pl.ds(cbase + tl * 128 + q * 16, 16)]
                    cr = lax.shift_right_logical(c32, 5) & 15
                    cvecs.append((c32, c32 + 16, cr))
                for d in range(D):
                    o = obase + (d // 8) * (TL * 1024) + tl * 1024 \
                        + (d % 8) * 128
                    for q in range(8):
                        base = cvecs[q][d // 16]
                        rot = (cvecs[q][2] + d) & 15
                        v = plsc.load_gather(wf_v, [base + rot])
                        out_v[pl.ds(o + q * 16, 16)] = v

            copies = [
                pltpu.async_copy(
                    out_v.at[pl.ds(obase + dg * (TL * 1024), TL * 1024)],
                    out_hbm.at[pl.ds(dg * (8 * E) + ch * (TL * 1024),
                                     TL * 1024)],
                    sem_out)
                for dg in range(4)
            ]

            @pl.when(t > 0)
            def _():
                for _ in range(4):
                    pltpu.make_async_copy(
                        out_v.at[pl.ds(0, TL * 1024)],
                        out_hbm.at[pl.ds(0, TL * 1024)],
                        sem_out).wait()

            pltpu.make_async_copy(
                c_hbm.at[pl.ds(0, S)], c_v.at[pl.ds(0, S)], sem_c).wait()
            return 0

        lax.fori_loop(0, cnt, chunk, 0, unroll=False)
        for _ in range(4):
            pltpu.make_async_copy(
                out_v.at[pl.ds(0, TL * 1024)],
                out_hbm.at[pl.ds(0, TL * 1024)],
                sem_out).wait()

    return pl.kernel(
        body,
        out_type=jax.ShapeDtypeStruct((D * E,), jnp.float32),
        mesh=mesh,
        compiler_params=pltpu.CompilerParams(needs_layout_passes=False),
        scratch_types=[
            pltpu.VMEM((NPAD * D,), jnp.float32),
            pltpu.VMEM((2 * S,), jnp.int32),
            pltpu.VMEM((2 * 4 * TL * 1024,), jnp.float32),
            pltpu.SemaphoreType.DMA,
            pltpu.SemaphoreType.DMA,
        ],
    )(wf_flat, c32)


def kernel(edge_features, W0, W1, W2):
    E = edge_features.shape[0]
    S = 1280
    assert E % S == 0
    ef = edge_features.astype(jnp.int32)
    c32 = (ef[:, 0] + ef[:, 1] * V0 + ef[:, 2] * (V0 * V1)) * D
    wf = _build_fused(W0, W1, W2).reshape(-1)
    out = _sc_gather(wf, c32, E, S)
    return out.reshape(4, E // 128, 8, 128).transpose(1, 3, 0, 2).reshape(E, D)
